# baseline (speedup 1.0000x reference)
import functools

import jax
import jax.numpy as jnp
from jax import lax
from jax.experimental import pallas as pl
from jax.experimental.pallas import tpu as pltpu
from jax.experimental.pallas import tpu_sc as plsc

N = 10000
E = 320000
D = 128

NC = 2
NS = 16
NW = NC * NS
EPW = 10240
EPAD = NW * EPW
CD = 80
NCHUNKD = (E // NW) // CD
C = 128
NCHUNK = 80
NPAD = 10240
RPT = NPAD // NS

BLK = 1000
GRID = N // BLK

_mesh = plsc.VectorSubcoreMesh(core_axis_name="c", subcore_axis_name="s")


@functools.partial(
    pl.kernel,
    out_type=jax.ShapeDtypeStruct((NC, NPAD, D), jnp.float32),
    mesh=_mesh,
    scratch_types=[
        pltpu.VMEM((NCHUNK, C), jnp.int32),
        pltpu.VMEM((NCHUNK, C), jnp.int32),
        pltpu.VMEM((C, D), jnp.float32),
        pltpu.VMEM_SHARED((NPAD, D), jnp.float32),
        pltpu.SemaphoreType.DMA,
    ],
)
def _sc_propagate(g_hbm, row_hbm, col_hbm, zero_hbm, out_hbm,
                  row_v, col_v, rows_v, acc, sem):
    c = lax.axis_index("c")
    s = lax.axis_index("s")
    wid = c * NS + s
    pltpu.sync_copy(zero_hbm, acc.at[pl.ds(s * RPT, RPT)])
    plsc.subcore_barrier()
    pltpu.sync_copy(row_hbm.at[wid], row_v)
    pltpu.sync_copy(col_hbm.at[wid], col_v)

    def chunk(j, carry):
        pltpu.async_copy(g_hbm.at[row_v.at[j]], rows_v, sem).wait()
        pltpu.sync_copy(rows_v, acc.at[col_v.at[j]], add=True)
        return carry

    lax.fori_loop(0, NCHUNK, chunk, 0)
    plsc.subcore_barrier()
    pltpu.sync_copy(acc.at[pl.ds(s * RPT, RPT)],
                    out_hbm.at[c, pl.ds(s * RPT, RPT)])


@functools.partial(
    pl.kernel,
    out_type=jax.ShapeDtypeStruct((NC, NPAD, D), jnp.float32),
    mesh=_mesh,
    scratch_types=[
        pltpu.VMEM((NCHUNKD, CD), jnp.int32),
        pltpu.VMEM((CD, D), jnp.float32),
        pltpu.VMEM_SHARED((NPAD, D), jnp.float32),
        pltpu.SemaphoreType.DMA,
    ],
)
def _sc_degree(col_hbm, ones_hbm, zero_hbm, out_hbm,
               col_v, ones_v, acc, sem):
    c = lax.axis_index("c")
    s = lax.axis_index("s")
    wid = c * NS + s
    pltpu.sync_copy(zero_hbm, acc.at[pl.ds(s * RPT, RPT)])
    plsc.subcore_barrier()
    pltpu.sync_copy(col_hbm.at[wid], col_v)
    pltpu.sync_copy(ones_hbm, ones_v)

    def chunk(j, carry):
        pltpu.sync_copy(ones_v, acc.at[col_v.at[j]], add=True)
        return carry

    lax.fori_loop(0, NCHUNKD, chunk, 0)
    plsc.subcore_barrier()
    pltpu.sync_copy(acc.at[pl.ds(s * RPT, RPT)],
                    out_hbm.at[c, pl.ds(s * RPT, RPT)])


def _row_spec(width=D):
    return pl.BlockSpec((BLK, width), lambda i: (i, 0))


def _w_spec():
    return pl.BlockSpec((D, D), lambda i: (0, 0))


def _b_spec(width=D):
    return pl.BlockSpec((1, width), lambda i: (0, 0))


def _p_spec(width=D):
    return pl.BlockSpec((NC, BLK, width), lambda i: (0, i, 0))


def _mm_t(x, w):
    return lax.dot_general(x, w, (((1,), (1,)), ((), ())),
                           preferred_element_type=jnp.float32)


def _tc_prep_body(pdeg_ref, x_ref, w0_ref, b0_ref, cw0_ref,
                  dis_ref, out_ref, g_ref):
    pdeg = pdeg_ref[...]
    deg = pdeg[0, :, 0:1] + pdeg[1, :, 0:1]
    dis = jnp.where(deg > 0, lax.rsqrt(jnp.where(deg > 0, deg, 1.0)), 0.0)
    x0 = jnp.maximum(_mm_t(x_ref[...], w0_ref[...]) + b0_ref[...], 0.0)
    dis_ref[...] = dis
    out_ref[...] = _mm_t(x0, cw0_ref[...])
    g_ref[...] = dis * x0


def _tc_prep(pdeg, x, w0, b0, cw0):
    return pl.pallas_call(
        _tc_prep_body,
        grid=(GRID,),
        in_specs=[_p_spec(), _row_spec(), _w_spec(), _b_spec(), _w_spec()],
        out_specs=[_row_spec(1), _row_spec(), _row_spec()],
        out_shape=[jax.ShapeDtypeStruct((N, 1), jnp.float32),
                   jax.ShapeDtypeStruct((N, D), jnp.float32),
                   jax.ShapeDtypeStruct((N, D), jnp.float32)],
    )(pdeg, x, w0, b0, cw0)


def _tc_step_body(p_ref, dis_ref, wk_ref, out_ref, o_ref, g_ref):
    p = p_ref[...]
    dis = dis_ref[...]
    h = dis * (p[0] + p[1])
    o_ref[...] = out_ref[...] + _mm_t(h, wk_ref[...])
    g_ref[...] = dis * h


def _tc_step(p, dis, wk, out):
    return pl.pallas_call(
        _tc_step_body,
        grid=(GRID,),
        in_specs=[_p_spec(), _row_spec(1), _w_spec(), _row_spec()],
        out_specs=[_row_spec(), _row_spec()],
        out_shape=[jax.ShapeDtypeStruct((N, D), jnp.float32),
                   jax.ShapeDtypeStruct((N, D), jnp.float32)],
    )(p, dis, wk, out)


def _tc_bridge_body(p_ref, dis_ref, w3_ref, out_ref, b_ref, cw0_ref,
                    o_ref, g_ref):
    p = p_ref[...]
    dis = dis_ref[...]
    h = dis * (p[0] + p[1])
    x1 = jnp.maximum(out_ref[...] + _mm_t(h, w3_ref[...]) + b_ref[...], 0.0)
    o_ref[...] = _mm_t(x1, cw0_ref[...])
    g_ref[...] = dis * x1


def _tc_bridge(p, dis, w3, out, b, cw0):
    return pl.pallas_call(
        _tc_bridge_body,
        grid=(GRID,),
        in_specs=[_p_spec(), _row_spec(1), _w_spec(), _row_spec(), _b_spec(),
                  _w_spec()],
        out_specs=[_row_spec(), _row_spec()],
        out_shape=[jax.ShapeDtypeStruct((N, D), jnp.float32),
                   jax.ShapeDtypeStruct((N, D), jnp.float32)],
    )(p, dis, w3, out, b, cw0)


def _tc_tail_body(p_ref, dis_ref, w3_ref, out_ref, b2_ref,
                  l1w_ref, l1b_ref, l2w_ref, l2b_ref, l3w_ref, l3b_ref,
                  y_ref):
    p = p_ref[...]
    dis = dis_ref[...]
    h = dis * (p[0] + p[1])
    x2 = jnp.maximum(out_ref[...] + _mm_t(h, w3_ref[...]) + b2_ref[...], 0.0)
    x3 = jnp.maximum(_mm_t(x2, l1w_ref[...]) + l1b_ref[...], 0.0)
    x4 = jnp.maximum(_mm_t(x3, l2w_ref[...]) + l2b_ref[...], 0.0)
    y_ref[...] = jnp.maximum(_mm_t(x4, l3w_ref[...])[:, 0:1] + l3b_ref[0, 0],
                             0.0)


def _tc_tail(p, dis, w3, out, b2, l1w, l1b, l2w, l2b, l3w, l3b):
    return pl.pallas_call(
        _tc_tail_body,
        grid=(GRID,),
        in_specs=[_p_spec(), _row_spec(1), _w_spec(), _row_spec(), _b_spec(),
                  _w_spec(), _b_spec(), _w_spec(), _b_spec(),
                  _w_spec(), _b_spec(1)],
        out_specs=_row_spec(1),
        out_shape=jax.ShapeDtypeStruct((N, 1), jnp.float32),
    )(p, dis, w3, out, b2, l1w, l1b, l2w, l2b, l3w, l3b)


def kernel(x_input, edge_index, batch, lin0_w, lin0_b, conv1_w, conv1_b,
           conv2_w, conv2_b, lin1_w, lin1_b, lin2_w, lin2_b, lin3_w, lin3_b):
    del batch
    pad = EPAD - E
    row3 = jnp.concatenate(
        [edge_index[0].astype(jnp.int32),
         jnp.zeros((pad,), jnp.int32)]).reshape(NW, NCHUNK, C)
    col3 = jnp.concatenate(
        [edge_index[1].astype(jnp.int32),
         jnp.full((pad,), NPAD - 1, jnp.int32)]).reshape(NW, NCHUNK, C)
    col3d = edge_index[1].reshape(NW, NCHUNKD, CD)
    zero_d = jnp.zeros((RPT, D), jnp.float32)
    ones_d = jnp.ones((CD, D), jnp.float32)

    pdeg = _sc_degree(col3d, ones_d, zero_d)
    b0 = lin0_b.reshape(1, D)
    b1 = conv1_b.reshape(1, D)
    b2 = conv2_b.reshape(1, D)

    dis, out, g = _tc_prep(pdeg, x_input, lin0_w, b0, conv1_w[0])
    for k in (1, 2):
        p = _sc_propagate(g, row3, col3, zero_d)
        out, g = _tc_step(p, dis, conv1_w[k], out)
    p = _sc_propagate(g, row3, col3, zero_d)
    out, g = _tc_bridge(p, dis, conv1_w[3], out, b1, conv2_w[0])
    for k in (1, 2):
        p = _sc_propagate(g, row3, col3, zero_d)
        out, g = _tc_step(p, dis, conv2_w[k], out)
    p = _sc_propagate(g, row3, col3, zero_d)
    l3w_pad = jnp.zeros((D, D), jnp.float32).at[0].set(lin3_w[0])
    y = _tc_tail(p, dis, conv2_w[3], out, b2,
                 lin1_w, lin1_b.reshape(1, D),
                 lin2_w, lin2_b.reshape(1, D),
                 l3w_pad, lin3_b.reshape(1, 1))
    return y


# speedup vs baseline: 3.0531x; 3.0531x over previous
import functools

import jax
import jax.numpy as jnp
from jax import lax
from jax.experimental import pallas as pl
from jax.experimental.pallas import tpu as pltpu
from jax.experimental.pallas import tpu_sc as plsc

N = 10000
E = 320000
D = 128

NC = 2
NS = 16
NW = NC * NS
EPW = 10240
EPAD = NW * EPW
CD = 80
NCHUNKD = (E // NW) // CD
C = 128
NCHUNK = 80
NPAD = 10240
RPT = NPAD // NS

BLK = 1000
GRID = N // BLK

_mesh = plsc.VectorSubcoreMesh(core_axis_name="c", subcore_axis_name="s")


@functools.partial(
    pl.kernel,
    out_type=jax.ShapeDtypeStruct((NC, NPAD, D), jnp.float32),
    mesh=_mesh,
    scratch_types=[
        pltpu.VMEM((NCHUNK, C), jnp.int32),
        pltpu.VMEM((NCHUNK, C), jnp.int32),
        pltpu.VMEM((C, D), jnp.float32),
        pltpu.VMEM_SHARED((NPAD, D), jnp.float32),
        pltpu.SemaphoreType.DMA,
    ],
)
def _sc_propagate(g_hbm, row_hbm, col_hbm, zero_hbm, out_hbm,
                  row_v, col_v, rows_v, acc, sem):
    c = lax.axis_index("c")
    s = lax.axis_index("s")
    wid = c * NS + s
    pltpu.sync_copy(zero_hbm, acc.at[pl.ds(s * RPT, RPT)])
    plsc.subcore_barrier()
    pltpu.sync_copy(row_hbm.at[wid], row_v)
    pltpu.sync_copy(col_hbm.at[wid], col_v)

    def chunk(j, carry):
        pltpu.async_copy(g_hbm.at[row_v.at[j]], rows_v, sem).wait()
        pltpu.sync_copy(rows_v, acc.at[col_v.at[j]], add=True)
        return carry

    lax.fori_loop(0, NCHUNK, chunk, 0)
    plsc.subcore_barrier()
    pltpu.sync_copy(acc.at[pl.ds(s * RPT, RPT)],
                    out_hbm.at[c, pl.ds(s * RPT, RPT)])


@functools.partial(
    pl.kernel,
    out_type=jax.ShapeDtypeStruct((NC, NPAD, D), jnp.float32),
    mesh=_mesh,
    scratch_types=[
        pltpu.VMEM((NCHUNKD, CD), jnp.int32),
        pltpu.VMEM((CD, D), jnp.float32),
        pltpu.VMEM_SHARED((NPAD, D), jnp.float32),
        pltpu.SemaphoreType.DMA,
    ],
)
def _sc_degree(col_hbm, ones_hbm, zero_hbm, out_hbm,
               col_v, ones_v, acc, sem):
    c = lax.axis_index("c")
    s = lax.axis_index("s")
    wid = c * NS + s
    pltpu.sync_copy(zero_hbm, acc.at[pl.ds(s * RPT, RPT)])
    plsc.subcore_barrier()
    pltpu.sync_copy(col_hbm.at[wid], col_v)
    pltpu.sync_copy(ones_hbm, ones_v)

    def chunk(j, carry):
        pltpu.sync_copy(ones_v, acc.at[col_v.at[j]], add=True)
        return carry

    lax.fori_loop(0, NCHUNKD, chunk, 0)
    plsc.subcore_barrier()
    pltpu.sync_copy(acc.at[pl.ds(s * RPT, RPT)],
                    out_hbm.at[c, pl.ds(s * RPT, RPT)])


def _row_spec(width=D):
    return pl.BlockSpec((BLK, width), lambda i: (i, 0))


def _w_spec():
    return pl.BlockSpec((D, D), lambda i: (0, 0))


def _b_spec(width=D):
    return pl.BlockSpec((1, width), lambda i: (0, 0))


def _p_spec(width=D):
    return pl.BlockSpec((NC, BLK, width), lambda i: (0, i, 0))


def _mm_t(x, w):
    return lax.dot_general(x, w, (((1,), (1,)), ((), ())),
                           preferred_element_type=jnp.float32)


def _tc_prep_body(pdeg_ref, x_ref, w0_ref, b0_ref, cw0_ref,
                  dis_ref, out_ref, g_ref):
    pdeg = pdeg_ref[...]
    deg = pdeg[0, :, 0:1] + pdeg[1, :, 0:1]
    dis = jnp.where(deg > 0, lax.rsqrt(jnp.where(deg > 0, deg, 1.0)), 0.0)
    x0 = jnp.maximum(_mm_t(x_ref[...], w0_ref[...]) + b0_ref[...], 0.0)
    dis_ref[...] = dis
    out_ref[...] = _mm_t(x0, cw0_ref[...])
    g_ref[...] = dis * x0


def _tc_prep(pdeg, x, w0, b0, cw0):
    return pl.pallas_call(
        _tc_prep_body,
        grid=(GRID,),
        in_specs=[_p_spec(), _row_spec(), _w_spec(), _b_spec(), _w_spec()],
        out_specs=[_row_spec(1), _row_spec(), _row_spec()],
        out_shape=[jax.ShapeDtypeStruct((N, 1), jnp.float32),
                   jax.ShapeDtypeStruct((N, D), jnp.float32),
                   jax.ShapeDtypeStruct((N, D), jnp.float32)],
    )(pdeg, x, w0, b0, cw0)


def _tc_step_body(p_ref, dis_ref, wk_ref, out_ref, o_ref, g_ref):
    p = p_ref[...]
    dis = dis_ref[...]
    h = dis * (p[0] + p[1])
    o_ref[...] = out_ref[...] + _mm_t(h, wk_ref[...])
    g_ref[...] = dis * h


def _tc_step(p, dis, wk, out):
    return pl.pallas_call(
        _tc_step_body,
        grid=(GRID,),
        in_specs=[_p_spec(), _row_spec(1), _w_spec(), _row_spec()],
        out_specs=[_row_spec(), _row_spec()],
        out_shape=[jax.ShapeDtypeStruct((N, D), jnp.float32),
                   jax.ShapeDtypeStruct((N, D), jnp.float32)],
    )(p, dis, wk, out)


def _tc_bridge_body(p_ref, dis_ref, w3_ref, out_ref, b_ref, cw0_ref,
                    o_ref, g_ref):
    p = p_ref[...]
    dis = dis_ref[...]
    h = dis * (p[0] + p[1])
    x1 = jnp.maximum(out_ref[...] + _mm_t(h, w3_ref[...]) + b_ref[...], 0.0)
    o_ref[...] = _mm_t(x1, cw0_ref[...])
    g_ref[...] = dis * x1


def _tc_bridge(p, dis, w3, out, b, cw0):
    return pl.pallas_call(
        _tc_bridge_body,
        grid=(GRID,),
        in_specs=[_p_spec(), _row_spec(1), _w_spec(), _row_spec(), _b_spec(),
                  _w_spec()],
        out_specs=[_row_spec(), _row_spec()],
        out_shape=[jax.ShapeDtypeStruct((N, D), jnp.float32),
                   jax.ShapeDtypeStruct((N, D), jnp.float32)],
    )(p, dis, w3, out, b, cw0)


def _tc_tail_body(p_ref, dis_ref, w3_ref, out_ref, b2_ref,
                  l1w_ref, l1b_ref, l2w_ref, l2b_ref, l3w_ref, l3b_ref,
                  y_ref):
    p = p_ref[...]
    dis = dis_ref[...]
    h = dis * (p[0] + p[1])
    x2 = jnp.maximum(out_ref[...] + _mm_t(h, w3_ref[...]) + b2_ref[...], 0.0)
    x3 = jnp.maximum(_mm_t(x2, l1w_ref[...]) + l1b_ref[...], 0.0)
    x4 = jnp.maximum(_mm_t(x3, l2w_ref[...]) + l2b_ref[...], 0.0)
    y_ref[...] = jnp.maximum(_mm_t(x4, l3w_ref[...])[:, 0:1] + l3b_ref[0, 0],
                             0.0)


def _tc_tail(p, dis, w3, out, b2, l1w, l1b, l2w, l2b, l3w, l3b):
    return pl.pallas_call(
        _tc_tail_body,
        grid=(GRID,),
        in_specs=[_p_spec(), _row_spec(1), _w_spec(), _row_spec(), _b_spec(),
                  _w_spec(), _b_spec(), _w_spec(), _b_spec(),
                  _w_spec(), _b_spec(1)],
        out_specs=_row_spec(1),
        out_shape=jax.ShapeDtypeStruct((N, 1), jnp.float32),
    )(p, dis, w3, out, b2, l1w, l1b, l2w, l2b, l3w, l3b)


def kernel(x_input, edge_index, batch, lin0_w, lin0_b, conv1_w, conv1_b,
           conv2_w, conv2_b, lin1_w, lin1_b, lin2_w, lin2_b, lin3_w, lin3_b):
    del batch
    pad = EPAD - E
    prow = jnp.arange(pad, dtype=jnp.int32) % N
    pcol = N + jnp.arange(pad, dtype=jnp.int32) % (NPAD - N)
    row3 = jnp.concatenate(
        [edge_index[0].astype(jnp.int32), prow]).reshape(NW, NCHUNK, C)
    col3 = jnp.concatenate(
        [edge_index[1].astype(jnp.int32), pcol]).reshape(NW, NCHUNK, C)
    col3d = edge_index[1].reshape(NW, NCHUNKD, CD)
    zero_d = jnp.zeros((RPT, D), jnp.float32)
    ones_d = jnp.ones((CD, D), jnp.float32)

    pdeg = _sc_degree(col3d, ones_d, zero_d)
    b0 = lin0_b.reshape(1, D)
    b1 = conv1_b.reshape(1, D)
    b2 = conv2_b.reshape(1, D)

    dis, out, g = _tc_prep(pdeg, x_input, lin0_w, b0, conv1_w[0])
    for k in (1, 2):
        p = _sc_propagate(g, row3, col3, zero_d)
        out, g = _tc_step(p, dis, conv1_w[k], out)
    p = _sc_propagate(g, row3, col3, zero_d)
    out, g = _tc_bridge(p, dis, conv1_w[3], out, b1, conv2_w[0])
    for k in (1, 2):
        p = _sc_propagate(g, row3, col3, zero_d)
        out, g = _tc_step(p, dis, conv2_w[k], out)
    p = _sc_propagate(g, row3, col3, zero_d)
    l3w_pad = jnp.zeros((D, D), jnp.float32).at[0].set(lin3_w[0])
    y = _tc_tail(p, dis, conv2_w[3], out, b2,
                 lin1_w, lin1_b.reshape(1, D),
                 lin2_w, lin2_b.reshape(1, D),
                 l3w_pad, lin3_b.reshape(1, 1))
    return y


# speedup vs baseline: 3.8903x; 1.2742x over previous
import functools

import jax
import jax.numpy as jnp
from jax import lax
from jax.experimental import pallas as pl
from jax.experimental.pallas import tpu as pltpu
from jax.experimental.pallas import tpu_sc as plsc

N = 10000
E = 320000
D = 128

NC = 2
NS = 16
NW = NC * NS
EPW = 10240
EPAD = NW * EPW
CD = 80
NCHUNKD = (E // NW) // CD
C = 128
NCHUNK = 80
NPAD = 10240
RPT = NPAD // NS
NB = 2
HC = C // 2
NU = 2 * NCHUNK

BLK = 1000
GRID = N // BLK

_mesh = plsc.VectorSubcoreMesh(core_axis_name="c", subcore_axis_name="s")


@functools.partial(
    pl.kernel,
    out_type=jax.ShapeDtypeStruct((NC, NPAD, D), jnp.float32),
    mesh=_mesh,
    scratch_types=[
        pltpu.VMEM((NCHUNK, C), jnp.int32),
        pltpu.VMEM((NU, HC), jnp.int32),
        pltpu.VMEM((NB, HC, D), jnp.float32),
        pltpu.VMEM_SHARED((NPAD, D), jnp.float32),
        pltpu.SemaphoreType.DMA,
        pltpu.SemaphoreType.DMA,
    ],
)
def _sc_propagate(g_hbm, row_hbm, col_hbm, zero_hbm, out_hbm,
                  row_v, col_v, rows_v, acc, gsem0, gsem1):
    c = lax.axis_index("c")
    s = lax.axis_index("s")
    wid = c * NS + s
    pltpu.sync_copy(zero_hbm, acc.at[pl.ds(s * RPT, RPT)])
    plsc.subcore_barrier()
    pltpu.sync_copy(row_hbm.at[wid], row_v)
    pltpu.sync_copy(col_hbm.at[wid], col_v)

    gsems = (gsem0, gsem1)

    def gather(j, b):
        pltpu.async_copy(g_hbm.at[row_v.at[j, pl.ds(b * HC, HC)]],
                         rows_v.at[b], gsems[b])

    def drain(b):
        pltpu.make_async_copy(g_hbm.at[row_v.at[0, pl.ds(0, HC)]],
                              rows_v.at[b], gsems[b]).wait()

    def scatter(u, b):
        pltpu.sync_copy(rows_v.at[b], acc.at[col_v.at[u]], add=True)

    for b in range(NB):
        gather(0, b)

    def body(i, carry):
        for b in range(NB):
            drain(b)
            scatter(NB * i + b, b)
            gather(i + 1, b)
        return carry

    lax.fori_loop(0, NCHUNK - 1, body, 0)
    for b in range(NB):
        drain(b)
        scatter(NU - NB + b, b)
    plsc.subcore_barrier()
    pltpu.sync_copy(acc.at[pl.ds(s * RPT, RPT)],
                    out_hbm.at[c, pl.ds(s * RPT, RPT)])


@functools.partial(
    pl.kernel,
    out_type=jax.ShapeDtypeStruct((NC, NPAD, D), jnp.float32),
    mesh=_mesh,
    scratch_types=[
        pltpu.VMEM((NCHUNKD, CD), jnp.int32),
        pltpu.VMEM((CD, D), jnp.float32),
        pltpu.VMEM_SHARED((NPAD, D), jnp.float32),
        pltpu.SemaphoreType.DMA,
    ],
)
def _sc_degree(col_hbm, ones_hbm, zero_hbm, out_hbm,
               col_v, ones_v, acc, sem):
    c = lax.axis_index("c")
    s = lax.axis_index("s")
    wid = c * NS + s
    pltpu.sync_copy(zero_hbm, acc.at[pl.ds(s * RPT, RPT)])
    plsc.subcore_barrier()
    pltpu.sync_copy(col_hbm.at[wid], col_v)
    pltpu.sync_copy(ones_hbm, ones_v)

    def chunk(j, carry):
        pltpu.sync_copy(ones_v, acc.at[col_v.at[j]], add=True)
        return carry

    lax.fori_loop(0, NCHUNKD, chunk, 0)
    plsc.subcore_barrier()
    pltpu.sync_copy(acc.at[pl.ds(s * RPT, RPT)],
                    out_hbm.at[c, pl.ds(s * RPT, RPT)])


def _row_spec(width=D):
    return pl.BlockSpec((BLK, width), lambda i: (i, 0))


def _w_spec():
    return pl.BlockSpec((D, D), lambda i: (0, 0))


def _b_spec(width=D):
    return pl.BlockSpec((1, width), lambda i: (0, 0))


def _p_spec(width=D):
    return pl.BlockSpec((NC, BLK, width), lambda i: (0, i, 0))


def _mm_t(x, w):
    return lax.dot_general(x, w, (((1,), (1,)), ((), ())),
                           preferred_element_type=jnp.float32)


def _tc_prep_body(pdeg_ref, x_ref, w0_ref, b0_ref, cw0_ref,
                  dis_ref, out_ref, g_ref):
    pdeg = pdeg_ref[...]
    deg = pdeg[0, :, 0:1] + pdeg[1, :, 0:1]
    dis = jnp.where(deg > 0, lax.rsqrt(jnp.where(deg > 0, deg, 1.0)), 0.0)
    x0 = jnp.maximum(_mm_t(x_ref[...], w0_ref[...]) + b0_ref[...], 0.0)
    dis_ref[...] = dis
    out_ref[...] = _mm_t(x0, cw0_ref[...])
    g_ref[...] = dis * x0


def _tc_prep(pdeg, x, w0, b0, cw0):
    return pl.pallas_call(
        _tc_prep_body,
        grid=(GRID,),
        in_specs=[_p_spec(), _row_spec(), _w_spec(), _b_spec(), _w_spec()],
        out_specs=[_row_spec(1), _row_spec(), _row_spec()],
        out_shape=[jax.ShapeDtypeStruct((N, 1), jnp.float32),
                   jax.ShapeDtypeStruct((N, D), jnp.float32),
                   jax.ShapeDtypeStruct((N, D), jnp.float32)],
    )(pdeg, x, w0, b0, cw0)


def _tc_step_body(p_ref, dis_ref, wk_ref, out_ref, o_ref, g_ref):
    p = p_ref[...]
    dis = dis_ref[...]
    h = dis * (p[0] + p[1])
    o_ref[...] = out_ref[...] + _mm_t(h, wk_ref[...])
    g_ref[...] = dis * h


def _tc_step(p, dis, wk, out):
    return pl.pallas_call(
        _tc_step_body,
        grid=(GRID,),
        in_specs=[_p_spec(), _row_spec(1), _w_spec(), _row_spec()],
        out_specs=[_row_spec(), _row_spec()],
        out_shape=[jax.ShapeDtypeStruct((N, D), jnp.float32),
                   jax.ShapeDtypeStruct((N, D), jnp.float32)],
    )(p, dis, wk, out)


def _tc_bridge_body(p_ref, dis_ref, w3_ref, out_ref, b_ref, cw0_ref,
                    o_ref, g_ref):
    p = p_ref[...]
    dis = dis_ref[...]
    h = dis * (p[0] + p[1])
    x1 = jnp.maximum(out_ref[...] + _mm_t(h, w3_ref[...]) + b_ref[...], 0.0)
    o_ref[...] = _mm_t(x1, cw0_ref[...])
    g_ref[...] = dis * x1


def _tc_bridge(p, dis, w3, out, b, cw0):
    return pl.pallas_call(
        _tc_bridge_body,
        grid=(GRID,),
        in_specs=[_p_spec(), _row_spec(1), _w_spec(), _row_spec(), _b_spec(),
                  _w_spec()],
        out_specs=[_row_spec(), _row_spec()],
        out_shape=[jax.ShapeDtypeStruct((N, D), jnp.float32),
                   jax.ShapeDtypeStruct((N, D), jnp.float32)],
    )(p, dis, w3, out, b, cw0)


def _tc_tail_body(p_ref, dis_ref, w3_ref, out_ref, b2_ref,
                  l1w_ref, l1b_ref, l2w_ref, l2b_ref, l3w_ref, l3b_ref,
                  y_ref):
    p = p_ref[...]
    dis = dis_ref[...]
    h = dis * (p[0] + p[1])
    x2 = jnp.maximum(out_ref[...] + _mm_t(h, w3_ref[...]) + b2_ref[...], 0.0)
    x3 = jnp.maximum(_mm_t(x2, l1w_ref[...]) + l1b_ref[...], 0.0)
    x4 = jnp.maximum(_mm_t(x3, l2w_ref[...]) + l2b_ref[...], 0.0)
    y_ref[...] = jnp.maximum(_mm_t(x4, l3w_ref[...])[:, 0:1] + l3b_ref[0, 0],
                             0.0)


def _tc_tail(p, dis, w3, out, b2, l1w, l1b, l2w, l2b, l3w, l3b):
    return pl.pallas_call(
        _tc_tail_body,
        grid=(GRID,),
        in_specs=[_p_spec(), _row_spec(1), _w_spec(), _row_spec(), _b_spec(),
                  _w_spec(), _b_spec(), _w_spec(), _b_spec(),
                  _w_spec(), _b_spec(1)],
        out_specs=_row_spec(1),
        out_shape=jax.ShapeDtypeStruct((N, 1), jnp.float32),
    )(p, dis, w3, out, b2, l1w, l1b, l2w, l2b, l3w, l3b)


def kernel(x_input, edge_index, batch, lin0_w, lin0_b, conv1_w, conv1_b,
           conv2_w, conv2_b, lin1_w, lin1_b, lin2_w, lin2_b, lin3_w, lin3_b):
    del batch
    pad = EPAD - E
    prow = jnp.arange(pad, dtype=jnp.int32) % N
    pcol = N + jnp.arange(pad, dtype=jnp.int32) % (NPAD - N)
    row3 = jnp.concatenate(
        [edge_index[0].astype(jnp.int32), prow]).reshape(NW, NCHUNK, C)
    col3 = jnp.concatenate(
        [edge_index[1].astype(jnp.int32), pcol]).reshape(NW, NU, HC)
    col3d = edge_index[1].reshape(NW, NCHUNKD, CD)
    zero_d = jnp.zeros((RPT, D), jnp.float32)
    ones_d = jnp.ones((CD, D), jnp.float32)

    pdeg = _sc_degree(col3d, ones_d, zero_d)
    b0 = lin0_b.reshape(1, D)
    b1 = conv1_b.reshape(1, D)
    b2 = conv2_b.reshape(1, D)

    dis, out, g = _tc_prep(pdeg, x_input, lin0_w, b0, conv1_w[0])
    for k in (1, 2):
        p = _sc_propagate(g, row3, col3, zero_d)
        out, g = _tc_step(p, dis, conv1_w[k], out)
    p = _sc_propagate(g, row3, col3, zero_d)
    out, g = _tc_bridge(p, dis, conv1_w[3], out, b1, conv2_w[0])
    for k in (1, 2):
        p = _sc_propagate(g, row3, col3, zero_d)
        out, g = _tc_step(p, dis, conv2_w[k], out)
    p = _sc_propagate(g, row3, col3, zero_d)
    l3w_pad = jnp.zeros((D, D), jnp.float32).at[0].set(lin3_w[0])
    y = _tc_tail(p, dis, conv2_w[3], out, b2,
                 lin1_w, lin1_b.reshape(1, D),
                 lin2_w, lin2_b.reshape(1, D),
                 l3w_pad, lin3_b.reshape(1, 1))
    return y
